# baseline (speedup 1.0000x reference)
import jax
import jax.numpy as jnp
from jax import lax
from jax.experimental import pallas as pl
from jax.experimental.pallas import tpu as pltpu
from jax.experimental.pallas import tpu_sc as plsc

_CLASS_NUM = 100000
_FEATURE_NUM = 64
_BATCH = 16384
_LAMBDAS = 2.0

_NC = 2
_NS = 16
_NW = _NC * _NS
_BPW = _BATCH // _NW
_HPW = _BPW // 2
_LROW = 128
_CNT_PER_TILE = _BATCH // _NS
_CPAD = 16 * 6272

_CB = 8192
_CSPLIT = 7 * _CB
_FSPLIT = _BATCH // 2


def _pack_body(a_ref, b_ref, o_ref):
  o_ref[...] = jnp.concatenate(
      [jnp.transpose(a_ref[...]), jnp.transpose(b_ref[...])], axis=-1)


def _count_body(lbl_hbm, w_hbm, table, lbl_cnt, ones_v, zeros_v,
                lbl_my, cnt_my, w_v):
  c = lax.axis_index("c")
  s = lax.axis_index("s")
  wid = s * _NC + c

  def fill_zeros(i, _):
    zeros_v[pl.ds(i * 16, 16)] = jnp.zeros((16,), jnp.float32)
    return 0
  lax.fori_loop(0, _CPAD // _NS // 16, fill_zeros, 0)

  def fill_ones(i, _):
    ones_v[pl.ds(i * 16, 16)] = jnp.ones((16,), jnp.float32)
    return 0
  lax.fori_loop(0, _CNT_PER_TILE // 16, fill_ones, 0)

  pltpu.sync_copy(zeros_v, table.at[pl.ds(s * (_CPAD // _NS), _CPAD // _NS)])
  plsc.subcore_barrier()

  pltpu.sync_copy(lbl_hbm.at[pl.ds(s * (_CNT_PER_TILE // _LROW),
                                   _CNT_PER_TILE // _LROW)], lbl_cnt)
  for j in range(_CNT_PER_TILE // _LROW):
    pltpu.sync_copy(ones_v.at[pl.ds(j * _LROW, _LROW)],
                    table.at[lbl_cnt.at[j]], add=True)
  plsc.subcore_barrier()

  pltpu.sync_copy(lbl_hbm.at[pl.ds(wid * (_BPW // _LROW),
                                   _BPW // _LROW)], lbl_my)
  for j in range(_BPW // _LROW):
    pltpu.sync_copy(table.at[lbl_my.at[j]],
                    cnt_my.at[pl.ds(j * _LROW, _LROW)])

  def invert(i, _):
    v = cnt_my[pl.ds(i * 16, 16)]
    w_v[pl.ds(i * 16, 16)] = 1.0 / v
    return 0
  lax.fori_loop(0, _BPW // 16, invert, 0)
  pltpu.sync_copy(w_v, w_hbm.at[pl.ds(wid * _BPW, _BPW)])


def _main_body(feat_hbm, lbl_hbm, cent_hbm, w_hbm, out_hbm,
               lbl_my, idx_my, off_v, w_v, cent_v, feat_v, acc_v,
               sem_c, sem_f, sem_w):
  c = lax.axis_index("c")
  s = lax.axis_index("s")
  wid = s * _NC + c

  pltpu.sync_copy(lbl_hbm.at[pl.ds(wid * (_BPW // _LROW),
                                   _BPW // _LROW)], lbl_my)

  def mk_idx(i, _):
    v = lbl_my[i >> 3, pl.ds((i & 7) * 16, 16)]
    hi = lax.shift_right_arithmetic(v - _CSPLIT, 31) + 1
    idx_my[i >> 3, pl.ds((i & 7) * 16, 16)] = v - hi * _CSPLIT
    off_v[pl.ds(i * 16, 16)] = lax.shift_left(hi, 6)
    return 0
  lax.fori_loop(0, _BPW // 16, mk_idx, 0)

  feat_dma = pltpu.async_copy(
      feat_hbm.at[pl.ds(wid * _HPW, _HPW)], feat_v, sem_f)
  w_dma = pltpu.async_copy(w_hbm.at[pl.ds(wid * _BPW, _BPW)], w_v, sem_w)
  cent_dmas = [
      pltpu.async_copy(cent_hbm.at[idx_my.at[j]],
                       cent_v.at[pl.ds(j * _LROW, _LROW)], sem_c)
      for j in range(_BPW // _LROW)
  ]
  feat_dma.wait()
  w_dma.wait()

  def make_group(fo, e0):
    def group(g, acc):
      wv16 = w_v[pl.ds(e0 + g * 16, 16)]
      off16 = off_v[pl.ds(e0 + g * 16, 16)]
      for i in range(16):
        r = e0 + g * 16 + i
        fr = g * 16 + i
        wv = jnp.full((16,), wv16[i], jnp.float32)
        o = off16[i]
        for q in range(_FEATURE_NUM // 16):
          f = feat_v[fr, pl.ds(fo + q * 16, 16)]
          cc = cent_v[r, pl.ds(o + q * 16, 16)]
          d = f - cc
          acc = acc + d * d * wv
      return acc
    return group

  acc = jnp.zeros((16,), jnp.float32)
  for j in range(_BPW // _LROW):
    cent_dmas[j].wait()
    e0 = j * _LROW
    if e0 < _HPW:
      acc = lax.fori_loop(e0 // 16, (e0 + _LROW) // 16, make_group(0, 0), acc)
    else:
      acc = lax.fori_loop((e0 - _HPW) // 16, (e0 - _HPW + _LROW) // 16,
                          make_group(_FEATURE_NUM, _HPW), acc)
  acc_v[...] = acc
  pltpu.sync_copy(acc_v, out_hbm.at[pl.ds(wid * 16, 16)])


@jax.jit
def kernel(feature, label, center):
  featT = feature.T
  centT = center.T
  lbl2d = (label.astype(jnp.int32)
           .reshape(2, _NW, _HPW)
           .transpose(1, 0, 2)
           .reshape(_BATCH // _LROW, _LROW))

  n_cb = _CSPLIT // _CB
  cent2 = pl.pallas_call(
      _pack_body,
      grid=(n_cb,),
      in_specs=[
          pl.BlockSpec((_FEATURE_NUM, _CB), lambda i: (0, i)),
          pl.BlockSpec((_FEATURE_NUM, _CB),
                       lambda i: (0, jnp.minimum(i + n_cb,
                                                 _CLASS_NUM // _CB))),
      ],
      out_specs=pl.BlockSpec((_CB, 2 * _FEATURE_NUM), lambda i: (i, 0)),
      out_shape=jax.ShapeDtypeStruct((_CSPLIT, 2 * _FEATURE_NUM),
                                     jnp.float32),
  )(centT, centT)

  n_fb = _FSPLIT // _CB
  feat2 = pl.pallas_call(
      _pack_body,
      grid=(n_fb,),
      in_specs=[
          pl.BlockSpec((_FEATURE_NUM, _CB), lambda i: (0, i)),
          pl.BlockSpec((_FEATURE_NUM, _CB), lambda i: (0, i + n_fb)),
      ],
      out_specs=pl.BlockSpec((_CB, 2 * _FEATURE_NUM), lambda i: (i, 0)),
      out_shape=jax.ShapeDtypeStruct((_FSPLIT, 2 * _FEATURE_NUM),
                                     jnp.float32),
  )(featT, featT)

  mesh = plsc.VectorSubcoreMesh(core_axis_name="c", subcore_axis_name="s")

  count_kern = pl.kernel(
      _count_body,
      out_type=jax.ShapeDtypeStruct((_BATCH,), jnp.float32),
      mesh=mesh,
      compiler_params=pltpu.CompilerParams(use_tc_tiling_on_sc=True),
      scratch_types=[
          pltpu.VMEM_SHARED((_CPAD,), jnp.float32),
          pltpu.VMEM((_CNT_PER_TILE // _LROW, _LROW), jnp.int32),
          pltpu.VMEM((_CNT_PER_TILE,), jnp.float32),
          pltpu.VMEM((_CPAD // _NS,), jnp.float32),
          pltpu.VMEM((_BPW // _LROW, _LROW), jnp.int32),
          pltpu.VMEM((_BPW,), jnp.float32),
          pltpu.VMEM((_BPW,), jnp.float32),
      ],
  )
  w = count_kern(lbl2d)

  main_kern = pl.kernel(
      _main_body,
      out_type=jax.ShapeDtypeStruct((_NW * 16,), jnp.float32),
      mesh=mesh,
      compiler_params=pltpu.CompilerParams(use_tc_tiling_on_sc=True),
      scratch_types=[
          pltpu.VMEM((_BPW // _LROW, _LROW), jnp.int32),
          pltpu.VMEM((_BPW // _LROW, _LROW), jnp.int32),
          pltpu.VMEM((_BPW,), jnp.int32),
          pltpu.VMEM((_BPW,), jnp.float32),
          pltpu.VMEM((_BPW, 2 * _FEATURE_NUM), jnp.float32),
          pltpu.VMEM((_HPW, 2 * _FEATURE_NUM), jnp.float32),
          pltpu.VMEM((16,), jnp.float32),
          pltpu.SemaphoreType.DMA,
          pltpu.SemaphoreType.DMA,
          pltpu.SemaphoreType.DMA,
      ],
  )
  partials = main_kern(feat2, lbl2d, cent2, w)
  return jnp.sum(partials) * (_LAMBDAS / 2.0 / _BATCH)


# speedup vs baseline: 1.0080x; 1.0080x over previous
import jax
import jax.numpy as jnp
from jax import lax
from jax.experimental import pallas as pl
from jax.experimental.pallas import tpu as pltpu
from jax.experimental.pallas import tpu_sc as plsc

_CLASS_NUM = 100000
_FEATURE_NUM = 64
_BATCH = 16384
_LAMBDAS = 2.0

_NC = 2
_NS = 16
_NW = _NC * _NS
_BPW = _BATCH // _NW
_HPW = _BPW // 2
_LROW = 128
_CNT_PER_TILE = _BATCH // _NS
_CPAD = 16 * 6272

_CB = 4096
_CSPLIT = 13 * _CB
_FSPLIT = _BATCH // 2


def _pack_body(a_ref, b_ref, o_ref):
  o_ref[...] = jnp.concatenate(
      [jnp.transpose(a_ref[...]), jnp.transpose(b_ref[...])], axis=-1)


def _count_body(lbl_hbm, w_hbm, table, lbl_cnt, ones_v, zeros_v,
                lbl_my, cnt_my, w_v):
  c = lax.axis_index("c")
  s = lax.axis_index("s")
  wid = s * _NC + c

  def fill_zeros(i, _):
    zeros_v[pl.ds(i * 16, 16)] = jnp.zeros((16,), jnp.float32)
    return 0
  lax.fori_loop(0, _CPAD // _NS // 16, fill_zeros, 0)

  def fill_ones(i, _):
    ones_v[pl.ds(i * 16, 16)] = jnp.ones((16,), jnp.float32)
    return 0
  lax.fori_loop(0, _CNT_PER_TILE // 16, fill_ones, 0)

  pltpu.sync_copy(zeros_v, table.at[pl.ds(s * (_CPAD // _NS), _CPAD // _NS)])
  plsc.subcore_barrier()

  pltpu.sync_copy(lbl_hbm.at[pl.ds(s * (_CNT_PER_TILE // _LROW),
                                   _CNT_PER_TILE // _LROW)], lbl_cnt)
  for j in range(_CNT_PER_TILE // _LROW):
    pltpu.sync_copy(ones_v.at[pl.ds(j * _LROW, _LROW)],
                    table.at[lbl_cnt.at[j]], add=True)
  plsc.subcore_barrier()

  pltpu.sync_copy(lbl_hbm.at[pl.ds(wid * (_BPW // _LROW),
                                   _BPW // _LROW)], lbl_my)
  for j in range(_BPW // _LROW):
    pltpu.sync_copy(table.at[lbl_my.at[j]],
                    cnt_my.at[pl.ds(j * _LROW, _LROW)])

  def invert(i, _):
    v = cnt_my[pl.ds(i * 16, 16)]
    w_v[pl.ds(i * 16, 16)] = 1.0 / v
    return 0
  lax.fori_loop(0, _BPW // 16, invert, 0)
  pltpu.sync_copy(w_v, w_hbm.at[pl.ds(wid * _BPW, _BPW)])


def _main_body(feat_hbm, lbl_hbm, cent_hbm, w_hbm, out_hbm,
               lbl_my, idx_my, off_v, w_v, cent_v, feat_v, acc_v,
               sem_c, sem_f, sem_w):
  c = lax.axis_index("c")
  s = lax.axis_index("s")
  wid = s * _NC + c

  pltpu.sync_copy(lbl_hbm.at[pl.ds(wid * (_BPW // _LROW),
                                   _BPW // _LROW)], lbl_my)

  def mk_idx(i, _):
    v = lbl_my[i >> 3, pl.ds((i & 7) * 16, 16)]
    hi = lax.shift_right_arithmetic(v - _CSPLIT, 31) + 1
    idx_my[i >> 3, pl.ds((i & 7) * 16, 16)] = v - hi * _CSPLIT
    off_v[pl.ds(i * 16, 16)] = lax.shift_left(hi, 6)
    return 0
  lax.fori_loop(0, _BPW // 16, mk_idx, 0)

  feat_dma = pltpu.async_copy(
      feat_hbm.at[pl.ds(wid * _HPW, _HPW)], feat_v, sem_f)
  w_dma = pltpu.async_copy(w_hbm.at[pl.ds(wid * _BPW, _BPW)], w_v, sem_w)
  cent_dmas = [
      pltpu.async_copy(cent_hbm.at[idx_my.at[j]],
                       cent_v.at[pl.ds(j * _LROW, _LROW)], sem_c)
      for j in range(_BPW // _LROW)
  ]
  feat_dma.wait()
  w_dma.wait()

  def make_group(fo, e0):
    def group(g, acc):
      wv16 = w_v[pl.ds(e0 + g * 16, 16)]
      off16 = off_v[pl.ds(e0 + g * 16, 16)]
      for i in range(16):
        r = e0 + g * 16 + i
        fr = g * 16 + i
        wv = jnp.full((16,), wv16[i], jnp.float32)
        o = off16[i]
        for q in range(_FEATURE_NUM // 16):
          f = feat_v[fr, pl.ds(fo + q * 16, 16)]
          cc = cent_v[r, pl.ds(o + q * 16, 16)]
          d = f - cc
          acc = acc + d * d * wv
      return acc
    return group

  acc = jnp.zeros((16,), jnp.float32)
  for j in range(_BPW // _LROW):
    cent_dmas[j].wait()
    e0 = j * _LROW
    if e0 < _HPW:
      acc = lax.fori_loop(e0 // 16, (e0 + _LROW) // 16, make_group(0, 0), acc)
    else:
      acc = lax.fori_loop((e0 - _HPW) // 16, (e0 - _HPW + _LROW) // 16,
                          make_group(_FEATURE_NUM, _HPW), acc)
  acc_v[...] = acc
  pltpu.sync_copy(acc_v, out_hbm.at[pl.ds(wid * 16, 16)])


@jax.jit
def kernel(feature, label, center):
  featT = feature.T
  centT = center.T
  lbl2d = (label.astype(jnp.int32)
           .reshape(2, _NW, _HPW)
           .transpose(1, 0, 2)
           .reshape(_BATCH // _LROW, _LROW))

  n_cb = _CSPLIT // _CB
  cent2 = pl.pallas_call(
      _pack_body,
      grid=(n_cb,),
      in_specs=[
          pl.BlockSpec((_FEATURE_NUM, _CB), lambda i: (0, i)),
          pl.BlockSpec((_FEATURE_NUM, _CB),
                       lambda i: (0, jnp.minimum(i + n_cb,
                                                 _CLASS_NUM // _CB))),
      ],
      out_specs=pl.BlockSpec((_CB, 2 * _FEATURE_NUM), lambda i: (i, 0)),
      out_shape=jax.ShapeDtypeStruct((_CSPLIT, 2 * _FEATURE_NUM),
                                     jnp.float32),
  )(centT, centT)

  n_fb = _FSPLIT // _CB
  feat2 = pl.pallas_call(
      _pack_body,
      grid=(n_fb,),
      in_specs=[
          pl.BlockSpec((_FEATURE_NUM, _CB), lambda i: (0, i)),
          pl.BlockSpec((_FEATURE_NUM, _CB), lambda i: (0, i + n_fb)),
      ],
      out_specs=pl.BlockSpec((_CB, 2 * _FEATURE_NUM), lambda i: (i, 0)),
      out_shape=jax.ShapeDtypeStruct((_FSPLIT, 2 * _FEATURE_NUM),
                                     jnp.float32),
  )(featT, featT)

  mesh = plsc.VectorSubcoreMesh(core_axis_name="c", subcore_axis_name="s")

  count_kern = pl.kernel(
      _count_body,
      out_type=jax.ShapeDtypeStruct((_BATCH,), jnp.float32),
      mesh=mesh,
      compiler_params=pltpu.CompilerParams(use_tc_tiling_on_sc=True),
      scratch_types=[
          pltpu.VMEM_SHARED((_CPAD,), jnp.float32),
          pltpu.VMEM((_CNT_PER_TILE // _LROW, _LROW), jnp.int32),
          pltpu.VMEM((_CNT_PER_TILE,), jnp.float32),
          pltpu.VMEM((_CPAD // _NS,), jnp.float32),
          pltpu.VMEM((_BPW // _LROW, _LROW), jnp.int32),
          pltpu.VMEM((_BPW,), jnp.float32),
          pltpu.VMEM((_BPW,), jnp.float32),
      ],
  )
  w = count_kern(lbl2d)

  main_kern = pl.kernel(
      _main_body,
      out_type=jax.ShapeDtypeStruct((_NW * 16,), jnp.float32),
      mesh=mesh,
      compiler_params=pltpu.CompilerParams(use_tc_tiling_on_sc=True),
      scratch_types=[
          pltpu.VMEM((_BPW // _LROW, _LROW), jnp.int32),
          pltpu.VMEM((_BPW // _LROW, _LROW), jnp.int32),
          pltpu.VMEM((_BPW,), jnp.int32),
          pltpu.VMEM((_BPW,), jnp.float32),
          pltpu.VMEM((_BPW, 2 * _FEATURE_NUM), jnp.float32),
          pltpu.VMEM((_HPW, 2 * _FEATURE_NUM), jnp.float32),
          pltpu.VMEM((16,), jnp.float32),
          pltpu.SemaphoreType.DMA,
          pltpu.SemaphoreType.DMA,
          pltpu.SemaphoreType.DMA,
      ],
  )
  partials = main_kern(feat2, lbl2d, cent2, w)
  return jnp.sum(partials) * (_LAMBDAS / 2.0 / _BATCH)
